# baseline (speedup 1.0000x reference)
import jax
import jax.numpy as jnp
from jax.experimental import pallas as pl
from jax.experimental.pallas import tpu as pltpu

jax.config.update("jax_enable_x64", True)

_N = 16
_B = 32
_BB = 16
_L2 = 1e-4
_SIGMA = 0.1
_ITERS = 18
_SFLOOR = 1e-25
_BIG = 1e30

import numpy as _np
_L2_HI = float(_np.float32(_L2))
_L2_LO = float(_np.float32(_L2 - _np.float64(_np.float32(_L2))))


def _qts(a, b):
    s = a + b
    return s, b - (s - a)


def _two_sum(a, b):
    s = a + b
    bb = s - a
    return s, (a - (s - bb)) + (b - bb)


def _split(a):
    c = jnp.float32(4097.0) * a
    hi = c - (c - a)
    return hi, a - hi


def _two_prod(a, b):
    p = a * b
    ah, al = _split(a)
    bh, bl = _split(b)
    return p, ((ah * bh - p) + ah * bl + al * bh) + al * bl


def df_add(a, b):
    sh, se = _two_sum(a[0], b[0])
    th, te = _two_sum(a[1], b[1])
    se = se + th
    sh, se = _qts(sh, se)
    return _qts(sh, se + te)


def df_neg(a):
    return (-a[0], -a[1])


def df_sub(a, b):
    return df_add(a, df_neg(b))


def df_mul(a, b):
    p, e = _two_prod(a[0], b[0])
    return _qts(p, e + (a[0] * b[1] + a[1] * b[0]))


def df_mul_f(a, b):
    p, e = _two_prod(a[0], b)
    return _qts(p, e + a[1] * b)


def df_div(a, b):
    q1 = a[0] / b[0]
    r = df_sub(a, df_mul_f(b, q1))
    q2 = r[0] / b[0]
    r = df_sub(r, df_mul_f(b, q2))
    q3 = r[0] / b[0]
    s, e = _qts(q1, q2)
    return df_add((s, e), (q3, jnp.zeros_like(q3)))


def df_recip(b):
    bh, bl = b
    q0 = 1.0 / bh
    q1 = q0 * (2.0 - q0 * bh)
    p, e = _two_prod(q1, bh)
    rh = ((1.0 - p) - e) - q1 * bl
    return _qts(q1, q1 * rh)


def df_sqrt(a):
    q = jnp.sqrt(a[0])
    p, e = _two_prod(q, q)
    r = df_sub(a, (p, e))
    return _qts(q, r[0] / (2.0 * q))


def df_from_f(x):
    return (x, jnp.zeros_like(x))


def df_where(m, a, b):
    return (jnp.where(m, a[0], b[0]), jnp.where(m, a[1], b[1]))


def df_sum_lanes(a):
    ah, al = a
    n = ah.shape[-1]
    while n > 1:
        h = n // 2
        ah, al = df_add((ah[..., :h], al[..., :h]), (ah[..., h:], al[..., h:]))
        n = h
    return ah, al


def df_sum_sub(a):
    ah, al = a
    n = ah.shape[1]
    while n > 1:
        h = n // 2
        ah, al = df_add((ah[:, :h, :], al[:, :h, :]), (ah[:, h:, :], al[:, h:, :]))
        n = h
    return ah, al


def _solve_newton(d1, dr, dc, r1, ra):
    f32 = jnp.float32
    N = _N
    ones_nn = jnp.ones_like(d1)
    L2df = (jnp.full_like(d1, _L2_HI), jnp.full_like(d1, _L2_LO))

    Dm = df_add(L2df, df_from_f(d1))
    E = df_recip(Dm)
    rs_k = df_sum_lanes(E)
    cs_k = df_sum_sub(E)
    invdr = df_recip(df_from_f(dr[:, :, None]))
    invdc = df_recip(df_from_f(dc[:, None, :]))
    K11 = df_add(invdr, rs_k)
    K22 = df_add(invdc, cs_k)
    invK11 = df_recip(K11)

    ir = jax.lax.broadcasted_iota(jnp.int32, (1, N, N), 1)
    ic = jax.lax.broadcasted_iota(jnp.int32, (1, N, N), 2)
    eye = ir == ic
    zero_nn = jnp.zeros_like(d1)
    W = df_mul(E, invK11)
    outer = df_mul((W[0][:, :, :, None], W[1][:, :, :, None]),
                   (E[0][:, :, None, :], E[1][:, :, None, :]))
    oh, ol = outer
    n = N
    while n > 1:
        h = n // 2
        oh, ol = df_add((oh[:, :h], ol[:, :h]), (oh[:, h:], ol[:, h:]))
        n = h
    S = df_add((jnp.where(eye, K22[0] + zero_nn, zero_nn),
                jnp.where(eye, K22[1] + zero_nn, zero_nn)),
               df_neg((oh[:, 0], ol[:, 0])))

    t1 = df_mul(E, df_from_f(r1))
    a1 = df_sum_lanes(t1)
    c1 = df_sum_sub(t1)
    aK1 = df_mul(a1, invK11)
    EtaK1 = df_sum_sub(df_mul(E, aK1))
    rhs1 = df_sub(c1, EtaK1)
    aK2 = df_mul(rs_k, invK11)
    EtaK2 = df_sum_sub(df_mul(E, aK2))
    rhs2 = df_sub(cs_k, EtaK2)

    th_h = jnp.concatenate([jnp.swapaxes(rhs1[0], 1, 2), jnp.swapaxes(rhs2[0], 1, 2)], axis=2)
    th_l = jnp.concatenate([jnp.swapaxes(rhs1[1], 1, 2), jnp.swapaxes(rhs2[1], 1, 2)], axis=2)
    th = (th_h, th_l)

    LTh = jnp.zeros_like(d1)
    LTl = jnp.zeros_like(d1)
    invd_h = jnp.zeros((d1.shape[0], 1, N), f32)
    invd_l = jnp.zeros_like(invd_h)
    ir2 = jax.lax.broadcasted_iota(jnp.int32, (1, N, 1), 1)
    ic2 = jax.lax.broadcasted_iota(jnp.int32, (1, 1, N), 2)
    zcol = jnp.zeros((1, N, 1), f32)
    zrow = jnp.zeros((1, 1, N), f32)
    for j in range(N):
        djj = (S[0][:, j:j + 1, j:j + 1], S[1][:, j:j + 1, j:j + 1])
        sq = df_sqrt(djj)
        inv = df_recip(sq)
        lane_j = ic2 == j
        invd_h = jnp.where(lane_j, inv[0], invd_h)
        invd_l = jnp.where(lane_j, inv[1], invd_l)
        col_s = df_mul((S[0][:, :, j:j + 1], S[1][:, :, j:j + 1]), inv)
        col_l = df_mul((S[0][:, j:j + 1, :], S[1][:, j:j + 1, :]), inv)
        cs_m = ir2 > j
        cl_m = ic2 > j
        col_s_z = (jnp.where(cs_m, col_s[0], zcol), jnp.where(cs_m, col_s[1], zcol))
        col_l_z = (jnp.where(cl_m, col_l[0], zrow), jnp.where(cl_m, col_l[1], zrow))
        row_j = ir == j
        LTh = jnp.where(row_j, col_l[0], LTh)
        LTl = jnp.where(row_j, col_l[1], LTl)
        if j + 1 < N:
            S = df_sub(S, df_mul(col_s_z, col_l_z))
        wj = df_mul(th, inv)
        th = df_where(ir2 == j, wj, th)
        roww = (th[0][:, j:j + 1, :], th[1][:, j:j + 1, :])
        th = df_sub(th, df_mul(col_s_z, roww))

    for j in reversed(range(N)):
        inv_j = (invd_h[:, :, j:j + 1], invd_l[:, :, j:j + 1])
        wj = df_mul(th, inv_j)
        th = df_where(ir2 == j, wj, th)
        if j > 0:
            roww = (th[0][:, j:j + 1, :], th[1][:, j:j + 1, :])
            cb_m = ir2 < j
            colLT = (jnp.where(cb_m, LTh[:, :, j:j + 1], zcol),
                     jnp.where(cb_m, LTl[:, :, j:j + 1], zcol))
            th = df_sub(th, df_mul(colLT, roww))

    outs = []
    for idx, (t, a) in enumerate(((t1, a1), (E, rs_k))):
        wc_s = (th[0][:, :, idx:idx + 1], th[1][:, :, idx:idx + 1])
        wc_l = (jnp.swapaxes(wc_s[0], 1, 2), jnp.swapaxes(wc_s[1], 1, 2))
        Ewc = df_sum_lanes(df_mul(E, wc_l))
        wr = df_mul(df_sub(a, Ewc), invK11)
        corr = df_add((wr[0] + jnp.zeros_like(d1), wr[1] + jnp.zeros_like(d1)),
                      (wc_l[0] + jnp.zeros_like(d1), wc_l[1] + jnp.zeros_like(d1)))
        outs.append(df_sub(t, df_mul(corr, E)))
    u, v = outs
    su = df_sum_sub(df_sum_lanes(u))
    sv = df_sum_sub(df_sum_lanes(v))
    dy = df_div(df_sub(df_from_f(ra), su), sv)
    dXdf = df_sub(df_neg(u), df_mul(v, (dy[0] + jnp.zeros_like(d1),
                                        dy[1] + jnp.zeros_like(d1))))
    return dXdf[0], dy[0]


def _ip_step(P, w1, w2, b, carry):
    X, s1, z1, sr, zr, sc, zc, y = carry
    f32 = jnp.float32
    NTOT = _N * _N + 2 * _N
    sum12 = lambda t: jnp.sum(t, axis=(1, 2), keepdims=True)
    sum1k = lambda t: jnp.sum(t, axis=1, keepdims=True)
    mu = (sum12(s1 * z1) + sum1k(sr * zr)[:, :, None]
          + sum1k(sc * zc)[:, :, None]) * (1.0 / NTOT)
    rx = _L2 * X + P - z1 + zr[:, :, None] + zc[:, None, :] + y
    rp1 = s1 - X
    rpr = jnp.sum(X, axis=2) + sr - w1
    rpc = jnp.sum(X, axis=1) + sc - w2
    ra = sum12(X) - b
    smu3 = _SIGMA * mu
    smu2 = smu3[:, :, 0]
    rs1 = s1 * z1 - smu3
    rsr = sr * zr - smu2
    rsc = sc * zc - smu2
    d1 = z1 / s1
    dr = zr / sr
    dc = zc / sc
    g1 = (z1 * rp1 - rs1) / s1
    gr = (zr * rpr - rsr) / sr
    gc = (zc * rpc - rsc) / sc
    r1 = rx - g1 + gr[:, :, None] + gc[:, None, :]

    H = d1.shape[0] // 2
    dX_a, dy_a = _solve_newton(d1[:H], dr[:H], dc[:H], r1[:H], ra[:H])
    dX_b, dy_b = _solve_newton(d1[H:], dr[H:], dc[H:], r1[H:], ra[H:])
    dX = jnp.concatenate([dX_a, dX_b], axis=0)
    dy = jnp.concatenate([dy_a, dy_b], axis=0)

    ds1 = dX - rp1
    dsr = -rpr - jnp.sum(dX, axis=2)
    dsc = -rpc - jnp.sum(dX, axis=1)
    dz1 = (-rs1 - z1 * ds1) / s1
    dzr = (-rsr - zr * dsr) / sr
    dzc = (-rsc - zc * dsc) / sc

    def ratio3(s, ds):
        r = jnp.where(ds < 0, -s / jnp.where(ds < 0, ds, -jnp.ones_like(ds)),
                      jnp.full_like(ds, _BIG))
        return jnp.min(r, axis=(1, 2), keepdims=True)

    def ratio2(s, ds):
        r = jnp.where(ds < 0, -s / jnp.where(ds < 0, ds, -jnp.ones_like(ds)),
                      jnp.full_like(ds, _BIG))
        return jnp.min(r, axis=1, keepdims=True)[:, :, None]

    a_s = jnp.minimum(ratio3(s1, ds1), jnp.minimum(ratio2(sr, dsr), ratio2(sc, dsc)))
    a_z = jnp.minimum(ratio3(z1, dz1), jnp.minimum(ratio2(zr, dzr), ratio2(zc, dzc)))
    alpha = 0.99 * jnp.minimum(jnp.ones_like(a_s), jnp.minimum(a_s, a_z))
    al2 = alpha[:, :, 0]
    floor = jnp.float32(_SFLOOR)
    cl = lambda t: jnp.maximum(t, floor)
    X = X + alpha * dX
    s1 = cl(s1 + alpha * ds1)
    z1 = cl(z1 + alpha * dz1)
    sr = cl(sr + al2 * dsr)
    zr = cl(zr + al2 * dzr)
    sc = cl(sc + al2 * dsc)
    zc = cl(zc + al2 * dzc)
    y = y + alpha * dy
    return (X, s1, z1, sr, zr, sc, zc, y)


def _emd_kernel(e1_ref, p1_ref, t1_ref, e2_ref, p2_ref, t2_ref, out_ref):
    f32 = jnp.float32
    e1 = e1_ref[...]
    p1 = p1_ref[...]
    w1 = t1_ref[...]
    e2 = e2_ref[...]
    p2 = p2_ref[...]
    w2 = t2_ref[...]

    de = -(e1[:, :, None] - e2[:, None, :]) + 1e-12
    dp = -(p1[:, :, None] - p2[:, None, :]) + 1e-12
    P = jnp.sqrt(de * de + dp * dp)

    sw1 = jnp.sum(w1, axis=1, keepdims=True)[:, :, None]
    sw2 = jnp.sum(w2, axis=1, keepdims=True)[:, :, None]
    b = jnp.minimum(sw1, sw2)

    BB = _BB
    X = jnp.zeros((BB, _N, _N), f32)
    s1 = jnp.ones((BB, _N, _N), f32)
    z1 = jnp.ones((BB, _N, _N), f32)
    sr = jnp.ones((BB, _N), f32)
    zr = jnp.ones((BB, _N), f32)
    sc = jnp.ones((BB, _N), f32)
    zc = jnp.ones((BB, _N), f32)
    y = jnp.zeros((BB, 1, 1), f32)

    carry = (X, s1, z1, sr, zr, sc, zc, y)
    carry = jax.lax.fori_loop(jnp.int32(0), jnp.int32(_ITERS),
                              lambda i, c: _ip_step(P, w1, w2, b, c), carry)
    X = carry[0]

    emd = jnp.sum(P * X, axis=(1, 2), keepdims=True) + jnp.abs(sw1 - sw2)
    out_ref[...] = emd[:, 0, 0][None, None, :]


def kernel(jets1, jets2):
    j1 = jets1.astype(jnp.float32)
    j2 = jets2.astype(jnp.float32)
    args = (j1[:, :, 0], j1[:, :, 1], j1[:, :, 2],
            j2[:, :, 0], j2[:, :, 1], j2[:, :, 2])
    out = pl.pallas_call(
        _emd_kernel,
        grid=(2,),
        in_specs=[pl.BlockSpec((_BB, _N), lambda i: (i, jnp.int32(0)))] * 6,
        out_specs=pl.BlockSpec((1, 1, _BB),
                               lambda i: (i, jnp.int32(0), jnp.int32(0))),
        out_shape=jax.ShapeDtypeStruct((2, 1, _BB), jnp.float32),
        compiler_params=pltpu.CompilerParams(dimension_semantics=("parallel",)),
    )(*args)
    return out.reshape(_B).astype(jnp.float64)


# speedup vs baseline: 1.1661x; 1.1661x over previous
import jax
import jax.numpy as jnp
from jax.experimental import pallas as pl
from jax.experimental.pallas import tpu as pltpu

jax.config.update("jax_enable_x64", True)

_N = 16
_B = 32
_BB = 16
_L2 = 1e-4
_SIGMA = 0.1
_ITERS = 18
_SFLOOR = 1e-25
_BIG = 1e30

import numpy as _np
_L2_HI = float(_np.float32(_L2))
_L2_LO = float(_np.float32(_L2 - _np.float64(_np.float32(_L2))))


def _qts(a, b):
    s = a + b
    return s, b - (s - a)


def _two_sum(a, b):
    s = a + b
    bb = s - a
    return s, (a - (s - bb)) + (b - bb)


def _split(a):
    c = jnp.float32(4097.0) * a
    hi = c - (c - a)
    return hi, a - hi


def _two_prod(a, b):
    p = a * b
    ah, al = _split(a)
    bh, bl = _split(b)
    return p, ((ah * bh - p) + ah * bl + al * bh) + al * bl


def df_add(a, b):
    sh, se = _two_sum(a[0], b[0])
    th, te = _two_sum(a[1], b[1])
    se = se + th
    sh, se = _qts(sh, se)
    return _qts(sh, se + te)


def df_neg(a):
    return (-a[0], -a[1])


def df_sub(a, b):
    return df_add(a, df_neg(b))


def df_mul(a, b):
    p, e = _two_prod(a[0], b[0])
    return _qts(p, e + (a[0] * b[1] + a[1] * b[0]))


def df_mul_f(a, b):
    p, e = _two_prod(a[0], b)
    return _qts(p, e + a[1] * b)


def df_div(a, b):
    q1 = a[0] / b[0]
    r = df_sub(a, df_mul_f(b, q1))
    q2 = r[0] / b[0]
    r = df_sub(r, df_mul_f(b, q2))
    q3 = r[0] / b[0]
    s, e = _qts(q1, q2)
    return df_add((s, e), (q3, jnp.zeros_like(q3)))


def df_recip(b):
    bh, bl = b
    q0 = 1.0 / bh
    q1 = q0 * (2.0 - q0 * bh)
    p, e = _two_prod(q1, bh)
    rh = ((1.0 - p) - e) - q1 * bl
    return _qts(q1, q1 * rh)


def df_sqrt(a):
    q = jnp.sqrt(a[0])
    p, e = _two_prod(q, q)
    r = df_sub(a, (p, e))
    return _qts(q, r[0] / (2.0 * q))


def df_from_f(x):
    return (x, jnp.zeros_like(x))


def df_where(m, a, b):
    return (jnp.where(m, a[0], b[0]), jnp.where(m, a[1], b[1]))


def df_sum_lanes(a):
    ah, al = a
    n = ah.shape[-1]
    while n > 1:
        h = n // 2
        ah, al = df_add((ah[..., :h], al[..., :h]), (ah[..., h:], al[..., h:]))
        n = h
    return ah, al


def df_sum_sub(a):
    ah, al = a
    n = ah.shape[1]
    while n > 1:
        h = n // 2
        ah, al = df_add((ah[:, :h, :], al[:, :h, :]), (ah[:, h:, :], al[:, h:, :]))
        n = h
    return ah, al


def _solve_newton(d1, dr, dc, r1, ra):
    f32 = jnp.float32
    N = _N
    ones_nn = jnp.ones_like(d1)
    L2df = (jnp.full_like(d1, _L2_HI), jnp.full_like(d1, _L2_LO))

    Dm = df_add(L2df, df_from_f(d1))
    E = df_recip(Dm)
    rs_k = df_sum_lanes(E)
    cs_k = df_sum_sub(E)
    invdr = df_recip(df_from_f(dr[:, :, None]))
    invdc = df_recip(df_from_f(dc[:, None, :]))
    K11 = df_add(invdr, rs_k)
    K22 = df_add(invdc, cs_k)
    invK11 = df_recip(K11)

    ET = (jnp.swapaxes(E[0], 1, 2), jnp.swapaxes(E[1], 1, 2))

    ir = jax.lax.broadcasted_iota(jnp.int32, (1, N, N), 1)
    ic = jax.lax.broadcasted_iota(jnp.int32, (1, N, N), 2)
    eye = ir == ic
    zero_nn = jnp.zeros_like(d1)
    S = (jnp.where(eye, K22[0] + zero_nn, zero_nn),
         jnp.where(eye, K22[1] + zero_nn, zero_nn))
    for i in range(N):
        f1 = df_mul((ET[0][:, :, i:i + 1], ET[1][:, :, i:i + 1]),
                    (invK11[0][:, i:i + 1, :], invK11[1][:, i:i + 1, :]))
        upd = df_mul(f1, (E[0][:, i:i + 1, :], E[1][:, i:i + 1, :]))
        S = df_sub(S, upd)

    t1 = df_mul(E, df_from_f(r1))
    a1 = df_sum_lanes(t1)
    c1 = df_sum_sub(t1)
    aK1 = df_mul(a1, invK11)
    EtaK1 = df_sum_sub(df_mul(E, aK1))
    rhs1 = df_sub(c1, EtaK1)
    aK2 = df_mul(rs_k, invK11)
    EtaK2 = df_sum_sub(df_mul(E, aK2))
    rhs2 = df_sub(cs_k, EtaK2)

    th_h = jnp.concatenate([jnp.swapaxes(rhs1[0], 1, 2), jnp.swapaxes(rhs2[0], 1, 2)], axis=2)
    th_l = jnp.concatenate([jnp.swapaxes(rhs1[1], 1, 2), jnp.swapaxes(rhs2[1], 1, 2)], axis=2)
    th = (th_h, th_l)

    LTh = jnp.zeros_like(d1)
    LTl = jnp.zeros_like(d1)
    invd_h = jnp.zeros((d1.shape[0], 1, N), f32)
    invd_l = jnp.zeros_like(invd_h)
    ir2 = jax.lax.broadcasted_iota(jnp.int32, (1, N, 1), 1)
    ic2 = jax.lax.broadcasted_iota(jnp.int32, (1, 1, N), 2)
    zcol = jnp.zeros((1, N, 1), f32)
    zrow = jnp.zeros((1, 1, N), f32)
    for j in range(N):
        djj = (S[0][:, j:j + 1, j:j + 1], S[1][:, j:j + 1, j:j + 1])
        sq = df_sqrt(djj)
        inv = df_recip(sq)
        lane_j = ic2 == j
        invd_h = jnp.where(lane_j, inv[0], invd_h)
        invd_l = jnp.where(lane_j, inv[1], invd_l)
        col_s = df_mul((S[0][:, :, j:j + 1], S[1][:, :, j:j + 1]), inv)
        col_l = df_mul((S[0][:, j:j + 1, :], S[1][:, j:j + 1, :]), inv)
        cs_m = ir2 > j
        cl_m = ic2 > j
        col_s_z = (jnp.where(cs_m, col_s[0], zcol), jnp.where(cs_m, col_s[1], zcol))
        col_l_z = (jnp.where(cl_m, col_l[0], zrow), jnp.where(cl_m, col_l[1], zrow))
        row_j = ir == j
        LTh = jnp.where(row_j, col_l[0], LTh)
        LTl = jnp.where(row_j, col_l[1], LTl)
        if j + 1 < N:
            S = df_sub(S, df_mul(col_s_z, col_l_z))
        wj = df_mul(th, inv)
        th = df_where(ir2 == j, wj, th)
        roww = (th[0][:, j:j + 1, :], th[1][:, j:j + 1, :])
        th = df_sub(th, df_mul(col_s_z, roww))

    for j in reversed(range(N)):
        inv_j = (invd_h[:, :, j:j + 1], invd_l[:, :, j:j + 1])
        wj = df_mul(th, inv_j)
        th = df_where(ir2 == j, wj, th)
        if j > 0:
            roww = (th[0][:, j:j + 1, :], th[1][:, j:j + 1, :])
            cb_m = ir2 < j
            colLT = (jnp.where(cb_m, LTh[:, :, j:j + 1], zcol),
                     jnp.where(cb_m, LTl[:, :, j:j + 1], zcol))
            th = df_sub(th, df_mul(colLT, roww))

    outs = []
    for idx, (t, a) in enumerate(((t1, a1), (E, rs_k))):
        wc_s = (th[0][:, :, idx:idx + 1], th[1][:, :, idx:idx + 1])
        wc_l = (jnp.swapaxes(wc_s[0], 1, 2), jnp.swapaxes(wc_s[1], 1, 2))
        Ewc = df_sum_lanes(df_mul(E, wc_l))
        wr = df_mul(df_sub(a, Ewc), invK11)
        corr = df_add((wr[0] + jnp.zeros_like(d1), wr[1] + jnp.zeros_like(d1)),
                      (wc_l[0] + jnp.zeros_like(d1), wc_l[1] + jnp.zeros_like(d1)))
        outs.append(df_sub(t, df_mul(corr, E)))
    u, v = outs
    su = df_sum_sub(df_sum_lanes(u))
    sv = df_sum_sub(df_sum_lanes(v))
    dy = df_div(df_sub(df_from_f(ra), su), sv)
    dXdf = df_sub(df_neg(u), df_mul(v, (dy[0] + jnp.zeros_like(d1),
                                        dy[1] + jnp.zeros_like(d1))))
    return dXdf[0], dy[0]


def _ip_step(P, w1, w2, b, carry):
    X, s1, z1, sr, zr, sc, zc, y = carry
    f32 = jnp.float32
    NTOT = _N * _N + 2 * _N
    sum12 = lambda t: jnp.sum(t, axis=(1, 2), keepdims=True)
    sum1k = lambda t: jnp.sum(t, axis=1, keepdims=True)
    mu = (sum12(s1 * z1) + sum1k(sr * zr)[:, :, None]
          + sum1k(sc * zc)[:, :, None]) * (1.0 / NTOT)
    rx = _L2 * X + P - z1 + zr[:, :, None] + zc[:, None, :] + y
    rp1 = s1 - X
    rpr = jnp.sum(X, axis=2) + sr - w1
    rpc = jnp.sum(X, axis=1) + sc - w2
    ra = sum12(X) - b
    smu3 = _SIGMA * mu
    smu2 = smu3[:, :, 0]
    rs1 = s1 * z1 - smu3
    rsr = sr * zr - smu2
    rsc = sc * zc - smu2
    d1 = z1 / s1
    dr = zr / sr
    dc = zc / sc
    g1 = (z1 * rp1 - rs1) / s1
    gr = (zr * rpr - rsr) / sr
    gc = (zc * rpc - rsc) / sc
    r1 = rx - g1 + gr[:, :, None] + gc[:, None, :]

    dX, dy = _solve_newton(d1, dr, dc, r1, ra)

    ds1 = dX - rp1
    dsr = -rpr - jnp.sum(dX, axis=2)
    dsc = -rpc - jnp.sum(dX, axis=1)
    dz1 = (-rs1 - z1 * ds1) / s1
    dzr = (-rsr - zr * dsr) / sr
    dzc = (-rsc - zc * dsc) / sc

    def ratio3(s, ds):
        r = jnp.where(ds < 0, -s / jnp.where(ds < 0, ds, -jnp.ones_like(ds)),
                      jnp.full_like(ds, _BIG))
        return jnp.min(r, axis=(1, 2), keepdims=True)

    def ratio2(s, ds):
        r = jnp.where(ds < 0, -s / jnp.where(ds < 0, ds, -jnp.ones_like(ds)),
                      jnp.full_like(ds, _BIG))
        return jnp.min(r, axis=1, keepdims=True)[:, :, None]

    a_s = jnp.minimum(ratio3(s1, ds1), jnp.minimum(ratio2(sr, dsr), ratio2(sc, dsc)))
    a_z = jnp.minimum(ratio3(z1, dz1), jnp.minimum(ratio2(zr, dzr), ratio2(zc, dzc)))
    alpha = 0.99 * jnp.minimum(jnp.ones_like(a_s), jnp.minimum(a_s, a_z))
    al2 = alpha[:, :, 0]
    floor = jnp.float32(_SFLOOR)
    cl = lambda t: jnp.maximum(t, floor)
    X = X + alpha * dX
    s1 = cl(s1 + alpha * ds1)
    z1 = cl(z1 + alpha * dz1)
    sr = cl(sr + al2 * dsr)
    zr = cl(zr + al2 * dzr)
    sc = cl(sc + al2 * dsc)
    zc = cl(zc + al2 * dzc)
    y = y + alpha * dy
    return (X, s1, z1, sr, zr, sc, zc, y)


def _emd_kernel(e1_ref, p1_ref, t1_ref, e2_ref, p2_ref, t2_ref, out_ref):
    f32 = jnp.float32
    e1 = e1_ref[...]
    p1 = p1_ref[...]
    w1 = t1_ref[...]
    e2 = e2_ref[...]
    p2 = p2_ref[...]
    w2 = t2_ref[...]

    de = -(e1[:, :, None] - e2[:, None, :]) + 1e-12
    dp = -(p1[:, :, None] - p2[:, None, :]) + 1e-12
    P = jnp.sqrt(de * de + dp * dp)

    sw1 = jnp.sum(w1, axis=1, keepdims=True)[:, :, None]
    sw2 = jnp.sum(w2, axis=1, keepdims=True)[:, :, None]
    b = jnp.minimum(sw1, sw2)

    BB = _BB
    X = jnp.zeros((BB, _N, _N), f32)
    s1 = jnp.ones((BB, _N, _N), f32)
    z1 = jnp.ones((BB, _N, _N), f32)
    sr = jnp.ones((BB, _N), f32)
    zr = jnp.ones((BB, _N), f32)
    sc = jnp.ones((BB, _N), f32)
    zc = jnp.ones((BB, _N), f32)
    y = jnp.zeros((BB, 1, 1), f32)

    carry = (X, s1, z1, sr, zr, sc, zc, y)
    carry = jax.lax.fori_loop(jnp.int32(0), jnp.int32(_ITERS),
                              lambda i, c: _ip_step(P, w1, w2, b, c), carry)
    X = carry[0]

    emd = jnp.sum(P * X, axis=(1, 2), keepdims=True) + jnp.abs(sw1 - sw2)
    out_ref[...] = emd[:, 0, 0][None, None, :]


def kernel(jets1, jets2):
    j1 = jets1.astype(jnp.float32)
    j2 = jets2.astype(jnp.float32)
    args = (j1[:, :, 0], j1[:, :, 1], j1[:, :, 2],
            j2[:, :, 0], j2[:, :, 1], j2[:, :, 2])
    out = pl.pallas_call(
        _emd_kernel,
        grid=(2,),
        in_specs=[pl.BlockSpec((_BB, _N), lambda i: (i, jnp.int32(0)))] * 6,
        out_specs=pl.BlockSpec((1, 1, _BB),
                               lambda i: (i, jnp.int32(0), jnp.int32(0))),
        out_shape=jax.ShapeDtypeStruct((2, 1, _BB), jnp.float32),
        compiler_params=pltpu.CompilerParams(dimension_semantics=("parallel",)),
    )(*args)
    return out.reshape(_B).astype(jnp.float64)
